# Initial kernel scaffold; baseline (speedup 1.0000x reference)
import jax
import jax.numpy as jnp
from jax.experimental import pallas as pl


def kernel(x, edge_index, W1s, W1d, a1s, a1d, b1, W2s, W2d, a2s, a2d, b2):
    raise NotImplementedError("write your pallas kernel here")


# speedup vs baseline: 17.7332x; 17.7332x over previous
import functools

import jax
import jax.numpy as jnp
from jax import lax
from jax.experimental import pallas as pl
from jax.experimental.pallas import tpu as pltpu
from jax.experimental.pallas import tpu_sc as plsc

N = 10000
D = 128
E = 320000
NC = 2
NS = 16
NW = NC * NS
EPW = E // NW
K = 80
NCHUNK = EPW // K
NPAD = 10240
RPT = NPAD // NS


def _prep_body(x_ref, ws_ref, asv_ref, wd_ref, adv_ref,
               xs_ref, as_ref, ad_ref, ms_ref, md_ref):
    i = pl.program_id(0)
    x = x_ref[...]
    xs = jnp.dot(x, ws_ref[...], preferred_element_type=jnp.float32)
    xs_ref[...] = xs
    a_s = jnp.dot(xs, asv_ref[...], preferred_element_type=jnp.float32)
    wdv = jnp.dot(wd_ref[...], adv_ref[...], preferred_element_type=jnp.float32)
    a_d = jnp.dot(x, wdv, preferred_element_type=jnp.float32)
    as_ref[...] = a_s
    ad_ref[...] = a_d

    @pl.when(i == 0)
    def _():
        ms_ref[0, 0] = -jnp.inf
        md_ref[0, 0] = -jnp.inf

    ms_ref[0, 0] = jnp.maximum(ms_ref[0, 0], jnp.max(a_s))
    md_ref[0, 0] = jnp.maximum(md_ref[0, 0], jnp.max(a_d))


def _prep(x, Ws, a_s, Wd, a_d):
    n = x.shape[0]
    bn = 1024 if n % 1024 == 0 else 1000
    grid = n // bn
    return pl.pallas_call(
        _prep_body,
        grid=(grid,),
        in_specs=[
            pl.BlockSpec((bn, D), lambda i: (i, 0)),
            pl.BlockSpec((D, D), lambda i: (0, 0)),
            pl.BlockSpec((D, 1), lambda i: (0, 0)),
            pl.BlockSpec((D, D), lambda i: (0, 0)),
            pl.BlockSpec((D, 1), lambda i: (0, 0)),
        ],
        out_specs=[
            pl.BlockSpec((bn, D), lambda i: (i, 0)),
            pl.BlockSpec((bn, 1), lambda i: (i, 0)),
            pl.BlockSpec((bn, 1), lambda i: (i, 0)),
            pl.BlockSpec((1, 1), lambda i: (0, 0), memory_space=pltpu.SMEM),
            pl.BlockSpec((1, 1), lambda i: (0, 0), memory_space=pltpu.SMEM),
        ],
        out_shape=[
            jax.ShapeDtypeStruct((n, D), jnp.float32),
            jax.ShapeDtypeStruct((n, 1), jnp.float32),
            jax.ShapeDtypeStruct((n, 1), jnp.float32),
            jax.ShapeDtypeStruct((1, 1), jnp.float32),
            jax.ShapeDtypeStruct((1, 1), jnp.float32),
        ],
    )(x, Ws, a_s.reshape(D, 1), Wd, a_d.reshape(D, 1))


def _sc_edge_body(xs_hbm, asrc_hbm, adst_hbm, src_hbm, dst_hbm, cvec_hbm,
                  outp_hbm, denp_hbm,
                  src_st, dst_st, asv_v, adv_v, exs_v,
                  rowbuf, zbuf, cvec_v, acc_sh, den_sh, sem):
    cid = lax.axis_index("c")
    sid = lax.axis_index("s")
    gw = cid * NS + sid
    ebase = gw * EPW
    row0 = sid * RPT

    zeros16 = jnp.zeros((16,), jnp.float32)

    def zrow(r, _):
        for j in range(8):
            rowbuf[r, pl.ds(j * 16, 16)] = zeros16
        return _
    lax.fori_loop(0, K, zrow, None)

    def zb(i, _):
        zbuf[pl.ds(i * 16, 16)] = zeros16
        return _
    lax.fori_loop(0, RPT // 16, zb, None)

    for kk in range(RPT // K):
        pltpu.sync_copy(rowbuf, acc_sh.at[pl.ds(row0 + kk * K, K), :])
    pltpu.sync_copy(zbuf, den_sh.at[pl.ds(row0, RPT)])

    pltpu.sync_copy(cvec_hbm, cvec_v)

    plsc.subcore_barrier()

    cv = cvec_v[...]

    def chunk(c, _):
        off = c * K
        pltpu.sync_copy(src_hbm.at[pl.ds(ebase + off, K)], src_st)
        pltpu.sync_copy(dst_hbm.at[pl.ds(ebase + off, K)], dst_st)
        pltpu.async_copy(asrc_hbm.at[src_st], asv_v, sem).wait()
        pltpu.async_copy(adst_hbm.at[dst_st], adv_v, sem).wait()
        for j in range(K // 16):
            a = asv_v[pl.ds(j * 16, 16)] + adv_v[pl.ds(j * 16, 16)]
            a = jnp.where(a >= 0.0, a, 0.2 * a) - cv
            exs_v[pl.ds(j * 16, 16)] = jnp.exp(a)
        pltpu.sync_copy(exs_v, den_sh.at[dst_st], add=True)
        pltpu.async_copy(xs_hbm.at[src_st], rowbuf, sem).wait()

        def rowblk(blk, _):
            ex16 = exs_v[pl.ds(blk * 16, 16)]
            for r in range(16):
                exr = jnp.full((16,), ex16[r], jnp.float32)
                rr = blk * 16 + r
                for j in range(8):
                    rowbuf[rr, pl.ds(j * 16, 16)] = (
                        rowbuf[rr, pl.ds(j * 16, 16)] * exr)
            return _
        lax.fori_loop(0, K // 16, rowblk, None)
        pltpu.sync_copy(rowbuf, acc_sh.at[dst_st], add=True)
        return _
    lax.fori_loop(0, NCHUNK, chunk, None)

    plsc.subcore_barrier()

    pltpu.sync_copy(acc_sh.at[pl.ds(row0, RPT), :],
                    outp_hbm.at[cid, pl.ds(row0, RPT), :])
    pltpu.sync_copy(den_sh.at[pl.ds(row0, RPT)],
                    denp_hbm.at[cid, pl.ds(row0, RPT)])


def _sc_edge(xs, asrc, adst, src, dst, cvec):
    mesh = plsc.VectorSubcoreMesh(core_axis_name="c", subcore_axis_name="s")
    f = pl.kernel(
        _sc_edge_body,
        out_type=[
            jax.ShapeDtypeStruct((NC, NPAD, D), jnp.float32),
            jax.ShapeDtypeStruct((NC, NPAD), jnp.float32),
        ],
        mesh=mesh,
        scratch_types=[
            pltpu.VMEM((K,), jnp.int32),
            pltpu.VMEM((K,), jnp.int32),
            pltpu.VMEM((K,), jnp.float32),
            pltpu.VMEM((K,), jnp.float32),
            pltpu.VMEM((K,), jnp.float32),
            pltpu.VMEM((K, D), jnp.float32),
            pltpu.VMEM((RPT,), jnp.float32),
            pltpu.VMEM((16,), jnp.float32),
            pltpu.VMEM_SHARED((NPAD, D), jnp.float32),
            pltpu.VMEM_SHARED((NPAD,), jnp.float32),
            pltpu.SemaphoreType.DMA,
        ],
    )
    return f(xs, asrc, adst, src, dst, cvec)


def _norm_body(relu, op_ref, dt_ref, b_ref, out_ref):
    p = op_ref[0] + op_ref[1]
    d = dt_ref[:, 0:1] + dt_ref[:, 1:2]
    o = p / (d + 1e-16) + b_ref[...]
    if relu:
        o = jnp.maximum(o, 0.0)
    out_ref[...] = o


def _norm(outp, denT, b, relu):
    bn = 1024
    grid = NPAD // bn
    return pl.pallas_call(
        functools.partial(_norm_body, relu),
        grid=(grid,),
        in_specs=[
            pl.BlockSpec((NC, bn, D), lambda i: (0, i, 0)),
            pl.BlockSpec((bn, NC), lambda i: (i, 0)),
            pl.BlockSpec((1, D), lambda i: (0, 0)),
        ],
        out_specs=pl.BlockSpec((bn, D), lambda i: (i, 0)),
        out_shape=jax.ShapeDtypeStruct((NPAD, D), jnp.float32),
    )(outp, denT, b.reshape(1, D))


def _layer(x, edge_src, edge_dst, Ws, Wd, a_s, a_d, b, relu):
    xs, asv, adv, ms, md = _prep(x, Ws, a_s, Wd, a_d)
    c = jnp.maximum(ms[0, 0] + md[0, 0], 0.0)
    cvec = jnp.full((16,), c, jnp.float32)
    outp, denp = _sc_edge(xs, asv.reshape(-1), adv.reshape(-1),
                          edge_src, edge_dst, cvec)
    return _norm(outp, denp.T, b, relu)


def kernel(x, edge_index, W1s, W1d, a1s, a1d, b1, W2s, W2d, a2s, a2d, b2):
    src = edge_index[0].astype(jnp.int32)
    dst = edge_index[1].astype(jnp.int32)
    h = _layer(x, src, dst, W1s, W1d, a1s, a1d, b1, relu=True)
    out = _layer(h, src, dst, W2s, W2d, a2s, a2d, b2, relu=False)
    return out[:N]


# speedup vs baseline: 21.3089x; 1.2016x over previous
import functools

import jax
import jax.numpy as jnp
from jax import lax
from jax.experimental import pallas as pl
from jax.experimental.pallas import tpu as pltpu
from jax.experimental.pallas import tpu_sc as plsc

N = 10000
D = 128
E = 320000
NC = 2
NS = 16
NW = NC * NS
EPW = E // NW
K = 80
NCHUNK = EPW // K
NPAD = 10240
RPT = NPAD // NS


def _prep_body(x_ref, ws_ref, asv_ref, wd_ref, adv_ref,
               xs_ref, as_ref, ad_ref, ms_ref, md_ref):
    i = pl.program_id(0)
    x = x_ref[...]
    xs = jnp.dot(x, ws_ref[...], preferred_element_type=jnp.float32)
    xs_ref[...] = xs
    a_s = jnp.dot(xs, asv_ref[...], preferred_element_type=jnp.float32)
    wdv = jnp.dot(wd_ref[...], adv_ref[...], preferred_element_type=jnp.float32)
    a_d = jnp.dot(x, wdv, preferred_element_type=jnp.float32)
    as_ref[...] = a_s
    ad_ref[...] = a_d

    @pl.when(i == 0)
    def _():
        ms_ref[0, 0] = -jnp.inf
        md_ref[0, 0] = -jnp.inf

    ms_ref[0, 0] = jnp.maximum(ms_ref[0, 0], jnp.max(a_s))
    md_ref[0, 0] = jnp.maximum(md_ref[0, 0], jnp.max(a_d))


def _prep(x, Ws, a_s, Wd, a_d):
    n = x.shape[0]
    bn = 1024 if n % 1024 == 0 else 1000
    grid = n // bn
    return pl.pallas_call(
        _prep_body,
        grid=(grid,),
        in_specs=[
            pl.BlockSpec((bn, D), lambda i: (i, 0)),
            pl.BlockSpec((D, D), lambda i: (0, 0)),
            pl.BlockSpec((D, 1), lambda i: (0, 0)),
            pl.BlockSpec((D, D), lambda i: (0, 0)),
            pl.BlockSpec((D, 1), lambda i: (0, 0)),
        ],
        out_specs=[
            pl.BlockSpec((bn, D), lambda i: (i, 0)),
            pl.BlockSpec((bn, 1), lambda i: (i, 0)),
            pl.BlockSpec((bn, 1), lambda i: (i, 0)),
            pl.BlockSpec((1, 1), lambda i: (0, 0), memory_space=pltpu.SMEM),
            pl.BlockSpec((1, 1), lambda i: (0, 0), memory_space=pltpu.SMEM),
        ],
        out_shape=[
            jax.ShapeDtypeStruct((n, D), jnp.float32),
            jax.ShapeDtypeStruct((n, 1), jnp.float32),
            jax.ShapeDtypeStruct((n, 1), jnp.float32),
            jax.ShapeDtypeStruct((1, 1), jnp.float32),
            jax.ShapeDtypeStruct((1, 1), jnp.float32),
        ],
    )(x, Ws, a_s.reshape(D, 1), Wd, a_d.reshape(D, 1))


def _sc_edge_body(xs_hbm, asrc_hbm, adst_hbm, src_hbm, dst_hbm, cvec_hbm,
                  outp_hbm, denp_hbm,
                  src_i, dst_i, asv_v, adv_v, exs_v,
                  rowbuf, zbuf, cvec_v, acc_sh, den_sh,
                  sem_si, sem_di, sem_ag, sem_dg, sem_row, sem_den, sem_acc):
    cid = lax.axis_index("c")
    sid = lax.axis_index("s")
    gw = cid * NS + sid
    ebase = gw * EPW
    row0 = sid * RPT

    zeros16 = jnp.zeros((16,), jnp.float32)

    def zrow(r, _):
        for j in range(8):
            rowbuf[0, r, pl.ds(j * 16, 16)] = zeros16
        return _
    lax.fori_loop(0, K, zrow, None)

    def zb(i, _):
        zbuf[pl.ds(i * 16, 16)] = zeros16
        return _
    lax.fori_loop(0, RPT // 16, zb, None)

    for kk in range(RPT // K):
        pltpu.sync_copy(rowbuf.at[0], acc_sh.at[pl.ds(row0 + kk * K, K), :])
    pltpu.sync_copy(zbuf, den_sh.at[pl.ds(row0, RPT)])

    pltpu.sync_copy(cvec_hbm, cvec_v)

    plsc.subcore_barrier()

    cv = cvec_v[...]

    def idx_issue(c):
        s = c % 4
        off = ebase + c * K
        pltpu.async_copy(src_hbm.at[pl.ds(off, K)], src_i.at[s],
                         sem_si.at[s])
        pltpu.async_copy(dst_hbm.at[pl.ds(off, K)], dst_i.at[s],
                         sem_di.at[s])

    def idx_wait(c):
        s = c % 4
        pltpu.make_async_copy(src_hbm.at[pl.ds(0, K)], src_i.at[s],
                              sem_si.at[s]).wait()
        pltpu.make_async_copy(dst_hbm.at[pl.ds(0, K)], dst_i.at[s],
                              sem_di.at[s]).wait()

    def gathers_issue(c):
        s = c % 4
        p = c % 2
        pltpu.async_copy(asrc_hbm.at[src_i.at[s]], asv_v.at[p],
                         sem_ag.at[p])
        pltpu.async_copy(adst_hbm.at[dst_i.at[s]], adv_v.at[p],
                         sem_dg.at[p])
        pltpu.async_copy(xs_hbm.at[src_i.at[s]], rowbuf.at[p],
                         sem_row.at[p])

    def alpha_wait(c):
        s = c % 4
        p = c % 2
        pltpu.make_async_copy(asrc_hbm.at[src_i.at[s]], asv_v.at[p],
                              sem_ag.at[p]).wait()
        pltpu.make_async_copy(adst_hbm.at[dst_i.at[s]], adv_v.at[p],
                              sem_dg.at[p]).wait()

    def row_wait(c):
        s = c % 4
        p = c % 2
        pltpu.make_async_copy(xs_hbm.at[src_i.at[s]], rowbuf.at[p],
                              sem_row.at[p]).wait()

    def den_wait(c):
        s = c % 4
        p = c % 2
        pltpu.make_async_copy(exs_v.at[p], den_sh.at[dst_i.at[s]],
                              sem_den.at[p]).wait()

    def acc_wait(c):
        s = c % 4
        p = c % 2
        pltpu.make_async_copy(rowbuf.at[p], acc_sh.at[dst_i.at[s]],
                              sem_acc.at[p]).wait()

    idx_issue(0)
    idx_issue(1)
    idx_wait(0)
    gathers_issue(0)

    def chunk(c, _):
        s = c % 4
        p = c % 2

        @pl.when(c + 2 < NCHUNK)
        def _():
            idx_issue(c + 2)

        @pl.when(c + 1 < NCHUNK)
        def _():
            idx_wait(c + 1)

            @pl.when(c >= 1)
            def _():
                acc_wait(c - 1)
                den_wait(c - 1)
            gathers_issue(c + 1)

        alpha_wait(c)
        for j in range(K // 16):
            a = (asv_v[p, pl.ds(j * 16, 16)]
                 + adv_v[p, pl.ds(j * 16, 16)])
            a = jnp.where(a >= 0.0, a, 0.2 * a) - cv
            exs_v[p, pl.ds(j * 16, 16)] = jnp.exp(a)
        pltpu.async_copy(exs_v.at[p], den_sh.at[dst_i.at[s]],
                         sem_den.at[p], add=True)

        row_wait(c)

        def rowblk(blk, _):
            ex16 = exs_v[p, pl.ds(blk * 16, 16)]
            for r in range(16):
                exr = jnp.full((16,), ex16[r], jnp.float32)
                rr = blk * 16 + r
                for j in range(8):
                    rowbuf[p, rr, pl.ds(j * 16, 16)] = (
                        rowbuf[p, rr, pl.ds(j * 16, 16)] * exr)
            return _
        lax.fori_loop(0, K // 16, rowblk, None)
        pltpu.async_copy(rowbuf.at[p], acc_sh.at[dst_i.at[s]],
                         sem_acc.at[p], add=True)
        return _
    lax.fori_loop(0, NCHUNK, chunk, None)

    acc_wait(NCHUNK - 2)
    den_wait(NCHUNK - 2)
    acc_wait(NCHUNK - 1)
    den_wait(NCHUNK - 1)

    plsc.subcore_barrier()

    pltpu.sync_copy(acc_sh.at[pl.ds(row0, RPT), :],
                    outp_hbm.at[cid, pl.ds(row0, RPT), :])
    pltpu.sync_copy(den_sh.at[pl.ds(row0, RPT)],
                    denp_hbm.at[cid, pl.ds(row0, RPT)])


def _sc_edge(xs, asrc, adst, src, dst, cvec):
    mesh = plsc.VectorSubcoreMesh(core_axis_name="c", subcore_axis_name="s")
    f = pl.kernel(
        _sc_edge_body,
        out_type=[
            jax.ShapeDtypeStruct((NC, NPAD, D), jnp.float32),
            jax.ShapeDtypeStruct((NC, NPAD), jnp.float32),
        ],
        mesh=mesh,
        scratch_types=[
            pltpu.VMEM((4, K), jnp.int32),
            pltpu.VMEM((4, K), jnp.int32),
            pltpu.VMEM((2, K), jnp.float32),
            pltpu.VMEM((2, K), jnp.float32),
            pltpu.VMEM((2, K), jnp.float32),
            pltpu.VMEM((2, K, D), jnp.float32),
            pltpu.VMEM((RPT,), jnp.float32),
            pltpu.VMEM((16,), jnp.float32),
            pltpu.VMEM_SHARED((NPAD, D), jnp.float32),
            pltpu.VMEM_SHARED((NPAD,), jnp.float32),
            pltpu.SemaphoreType.DMA((4,)),
            pltpu.SemaphoreType.DMA((4,)),
            pltpu.SemaphoreType.DMA((2,)),
            pltpu.SemaphoreType.DMA((2,)),
            pltpu.SemaphoreType.DMA((2,)),
            pltpu.SemaphoreType.DMA((2,)),
            pltpu.SemaphoreType.DMA((2,)),
        ],
    )
    return f(xs, asrc, adst, src, dst, cvec)


def _norm_body(relu, op_ref, dt_ref, b_ref, out_ref):
    p = op_ref[0] + op_ref[1]
    d = dt_ref[:, 0:1] + dt_ref[:, 1:2]
    o = p / (d + 1e-16) + b_ref[...]
    if relu:
        o = jnp.maximum(o, 0.0)
    out_ref[...] = o


def _norm(outp, denT, b, relu):
    bn = 1024
    grid = NPAD // bn
    return pl.pallas_call(
        functools.partial(_norm_body, relu),
        grid=(grid,),
        in_specs=[
            pl.BlockSpec((NC, bn, D), lambda i: (0, i, 0)),
            pl.BlockSpec((bn, NC), lambda i: (i, 0)),
            pl.BlockSpec((1, D), lambda i: (0, 0)),
        ],
        out_specs=pl.BlockSpec((bn, D), lambda i: (i, 0)),
        out_shape=jax.ShapeDtypeStruct((NPAD, D), jnp.float32),
    )(outp, denT, b.reshape(1, D))


def _layer(x, edge_src, edge_dst, Ws, Wd, a_s, a_d, b, relu):
    xs, asv, adv, ms, md = _prep(x, Ws, a_s, Wd, a_d)
    c = jnp.maximum(ms[0, 0] + md[0, 0], 0.0)
    cvec = jnp.full((16,), c, jnp.float32)
    outp, denp = _sc_edge(xs, asv.reshape(-1), adv.reshape(-1),
                          edge_src, edge_dst, cvec)
    return _norm(outp, denp.T, b, relu)


def kernel(x, edge_index, W1s, W1d, a1s, a1d, b1, W2s, W2d, a2s, a2d, b2):
    src = edge_index[0].astype(jnp.int32)
    dst = edge_index[1].astype(jnp.int32)
    h = _layer(x, src, dst, W1s, W1d, a1s, a1d, b1, relu=True)
    out = _layer(h, src, dst, W2s, W2d, a2s, a2d, b2, relu=False)
    return out[:N]
